# baseline (speedup 1.0000x reference)
import functools

import jax
import jax.numpy as jnp
from jax import lax
from jax.experimental import pallas as pl
from jax.experimental.pallas import tpu as pltpu
from jax.experimental.pallas import tpu_sc as plsc

_ALPHA = 0.25
_LOSS_WEIGHT = 1.0

_NC = 2
_NS = 16
_NW = _NC * _NS
_LANES = 16
_UNROLL = 4
_NCHUNK = 4


def _focal_vec(p, tgt):
    t = tgt == 0
    nonneg = p >= 0
    ap = jnp.abs(p)
    e = jnp.exp(-ap)
    r = 1.0 / (1.0 + e)
    er = e * r
    z = e / (e + 2.0)
    w = z * z
    poly = 1.0 + w * (1.0 / 3.0 + w * (1.0 / 5.0 + w * (1.0 / 7.0 + w * (1.0 / 9.0))))
    l1p = 2.0 * z * poly
    q = jnp.where(t, -p, p)
    bce = jnp.maximum(q, 0.0) + l1p
    s = jnp.where(nonneg, r, er)
    pt = jnp.where(t, 1.0 - s, s)
    af = jnp.where(t, _ALPHA, 1.0 - _ALPHA)
    return bce * af * pt * pt


def _sc_partial_sums(predf, target):
    n = predf.shape[0]
    per_w = n // _NW
    chunk = per_w // _NCHUNK
    vecs = chunk // (_UNROLL * _LANES)
    mesh = plsc.VectorSubcoreMesh(core_axis_name="c", subcore_axis_name="s")

    @functools.partial(
        pl.kernel,
        mesh=mesh,
        out_type=jax.ShapeDtypeStruct((_NW, _LANES), jnp.float32),
        scratch_types=[
            pltpu.VMEM((2, chunk), jnp.float32),
            pltpu.VMEM((2, chunk), jnp.int32),
            pltpu.VMEM((_LANES,), jnp.float32),
            pltpu.SemaphoreType.DMA,
            pltpu.SemaphoreType.DMA,
            pltpu.SemaphoreType.DMA,
            pltpu.SemaphoreType.DMA,
        ],
    )
    def sc_loss(pred_hbm, tgt_hbm, out_hbm, pred_v, tgt_v, acc_v,
                sp0, sp1, st0, st1):
        wid = lax.axis_index("s") * _NC + lax.axis_index("c")
        base = wid * per_w
        sem_p = (sp0, sp1)
        sem_t = (st0, st1)

        def issue(g, slot):
            off = base + g * chunk
            cp = pltpu.async_copy(pred_hbm.at[pl.ds(off, chunk)],
                                  pred_v.at[slot], sem_p[slot])
            ct = pltpu.async_copy(tgt_hbm.at[pl.ds(off, chunk)],
                                  tgt_v.at[slot], sem_t[slot])
            return cp, ct

        zero = jnp.zeros((_LANES,), jnp.float32)
        accs = (zero,) * _UNROLL
        pending = issue(0, 0)
        for g in range(_NCHUNK):
            slot = g % 2
            pending[0].wait()
            pending[1].wait()
            if g + 1 < _NCHUNK:
                pending = issue(g + 1, 1 - slot)

            def body(i, a, slot=slot):
                b = i * (_UNROLL * _LANES)
                out = []
                for k in range(_UNROLL):
                    p = pred_v.at[slot][pl.ds(b + k * _LANES, _LANES)]
                    tg = tgt_v.at[slot][pl.ds(b + k * _LANES, _LANES)]
                    out.append(a[k] + _focal_vec(p, tg))
                return tuple(out)

            accs = lax.fori_loop(0, vecs, body, accs)

        acc_v[...] = (accs[0] + accs[1]) + (accs[2] + accs[3])
        pltpu.sync_copy(acc_v, out_hbm.at[wid])

    return sc_loss(predf, target)


def kernel(pred, target):
    n = pred.shape[0]
    predf = pred.reshape(n)
    partials = _sc_partial_sums(predf, target)
    return _LOSS_WEIGHT * (jnp.sum(partials) / n)
